# baseline (speedup 1.0000x reference)
import jax
import jax.numpy as jnp
from jax.experimental import pallas as pl
from jax.experimental.pallas import tpu as pltpu

CHUNK = 512
NBUF = 4


def _router_body(x_hbm, w_ref, b_ref, o_ref, buf, sems):
    rows = x_hbm.shape[0]
    nchunks = rows // CHUNK

    def copy(i, slot):
        return pltpu.make_async_copy(
            x_hbm.at[pl.ds(i * CHUNK, CHUNK), :], buf.at[slot], sems.at[slot]
        )

    for s in range(min(NBUF, nchunks)):
        copy(s, s).start()

    for i in range(nchunks):
        slot = i % NBUF
        copy(i, slot).wait()
        logits = jax.lax.dot_general(
            buf[slot], w_ref[...],
            dimension_numbers=(((1,), (1,)), ((), ())),
            preferred_element_type=jnp.float32,
        ) + b_ref[...]
        m = jnp.max(logits, axis=-1, keepdims=True)
        e = jnp.exp(logits - m)
        o_ref[pl.ds(i * CHUNK, CHUNK), :] = e / jnp.sum(e, axis=-1, keepdims=True)
        nxt = i + NBUF
        if nxt < nchunks:
            copy(nxt, slot).start()


def kernel(x, W, b):
    B, T, D = x.shape
    E = W.shape[0]
    rows = B * T
    x2 = x.reshape(rows, D)
    out = pl.pallas_call(
        _router_body,
        in_specs=[
            pl.BlockSpec(memory_space=pltpu.MemorySpace.HBM),
            pl.BlockSpec(memory_space=pltpu.MemorySpace.VMEM),
            pl.BlockSpec(memory_space=pltpu.MemorySpace.VMEM),
        ],
        out_specs=pl.BlockSpec(memory_space=pltpu.MemorySpace.VMEM),
        out_shape=jax.ShapeDtypeStruct((rows, E), jnp.float32),
        scratch_shapes=[
            pltpu.VMEM((NBUF, CHUNK, D), jnp.float32),
            pltpu.SemaphoreType.DMA((NBUF,)),
        ],
    )(x2, W, b)
    return out.reshape(B, T, E)
